# Initial kernel scaffold; baseline (speedup 1.0000x reference)
import jax
import jax.numpy as jnp
from jax.experimental import pallas as pl


def kernel(graph, node_feats, edge_feats, params):
    raise NotImplementedError("write your pallas kernel here")


# speedup vs baseline: 8.1290x; 8.1290x over previous
import functools

import jax
import jax.numpy as jnp
from jax import lax
from jax.experimental import pallas as pl
from jax.experimental.pallas import tpu as pltpu
from jax.experimental.pallas import tpu_sc as plsc

N = 10000
E = 320000
DN = 128
DE = 16
H = 64
L = 4

F32 = jnp.float32

NSC = 2
NTILE = 16
NW = NSC * NTILE
EPT = E // NW
CH = 80
NCHUNK = EPT // CH
NP = 10240
RPT = NP // NTILE
SW = 16


def _matmul_bias_body(x_ref, w_ref, b_ref, o_ref):
    o_ref[...] = (
        jnp.dot(x_ref[...], w_ref[...], preferred_element_type=F32) + b_ref[...]
    )


def _tc_matmul_bias(x, w, b2d, block_rows):
    rows, k = x.shape
    n = w.shape[1]
    grid = rows // block_rows
    return pl.pallas_call(
        _matmul_bias_body,
        grid=(grid,),
        in_specs=[
            pl.BlockSpec((block_rows, k), lambda i: (i, 0)),
            pl.BlockSpec((k, n), lambda i: (0, 0)),
            pl.BlockSpec((1, n), lambda i: (0, 0)),
        ],
        out_specs=pl.BlockSpec((block_rows, n), lambda i: (i, 0)),
        out_shape=jax.ShapeDtypeStruct((rows, n), F32),
    )(x, w, b2d)


def _node_proj_body(h_ref, wmt_ref, wat_ref, hm_ref, ha_ref):
    h = h_ref[...]
    hm_ref[...] = jnp.dot(h, wmt_ref[...], preferred_element_type=F32)
    ha_ref[...] = (
        jnp.sum(h * wat_ref[0:1, :], axis=1, keepdims=True) + wat_ref[1:2, 0:1]
    )


def _tc_node_proj(h, wmt_pad, wat_plus_b):
    blk = 1000
    return pl.pallas_call(
        _node_proj_body,
        grid=(N // blk,),
        in_specs=[
            pl.BlockSpec((blk, H), lambda i: (i, 0)),
            pl.BlockSpec((H, 2 * H), lambda i: (0, 0)),
            pl.BlockSpec((2, H), lambda i: (0, 0)),
        ],
        out_specs=[
            pl.BlockSpec((blk, 2 * H), lambda i: (i, 0)),
            pl.BlockSpec((blk, 1), lambda i: (i, 0)),
        ],
        out_shape=[
            jax.ShapeDtypeStruct((N, 2 * H), F32),
            jax.ShapeDtypeStruct((N, 1), F32),
        ],
    )(h, wmt_pad, wat_plus_b)


def _gru_body(c_ref, h_ref, wxzr_ref, whzr_ref, bzr_ref, wxn_ref,
              whn_ref, bn_ref, o_ref):
    acc = c_ref[0] + c_ref[1]
    num = acc[:, :H]
    den = acc[:, H:H + 1]
    ctx = num / (den + 1e-16)
    ctx = jnp.where(ctx > 0, ctx, jnp.exp(jnp.minimum(ctx, 0.0)) - 1.0)
    h = h_ref[...]
    zr = jax.nn.sigmoid(
        jnp.dot(ctx, wxzr_ref[...], preferred_element_type=F32)
        + jnp.dot(h, whzr_ref[...], preferred_element_type=F32)
        + bzr_ref[...]
    )
    z = zr[:, :H]
    r = zr[:, H:]
    n = jnp.tanh(
        jnp.dot(ctx, wxn_ref[...], preferred_element_type=F32)
        + jnp.dot(r * h, whn_ref[...], preferred_element_type=F32)
        + bn_ref[...]
    )
    o_ref[...] = (1.0 - z) * n + z * h


def _tc_gru(c2, h, wxzr, whzr, bzr, wxn, whn, bn):
    blk = 1000
    return pl.pallas_call(
        _gru_body,
        grid=(N // blk,),
        in_specs=[
            pl.BlockSpec((2, blk, 2 * H), lambda i: (0, i, 0)),
            pl.BlockSpec((blk, H), lambda i: (i, 0)),
            pl.BlockSpec((H, 2 * H), lambda i: (0, 0)),
            pl.BlockSpec((H, 2 * H), lambda i: (0, 0)),
            pl.BlockSpec((1, 2 * H), lambda i: (0, 0)),
            pl.BlockSpec((H, H), lambda i: (0, 0)),
            pl.BlockSpec((H, H), lambda i: (0, 0)),
            pl.BlockSpec((1, H), lambda i: (0, 0)),
        ],
        out_specs=pl.BlockSpec((blk, H), lambda i: (i, 0)),
        out_shape=jax.ShapeDtypeStruct((N, H), F32),
    )(c2, h, wxzr, whzr, bzr[None, :], wxn, whn, bn[None, :])


def _readout_body(h_ref, wg_ref, rwxzr_ref, rwhzr_ref, rbzr_ref, rwxn_ref,
                  rwhn_ref, rbn_ref, w1_ref, b1_ref, w2_ref, b2_ref, o_ref):
    h = h_ref[...]
    g = jnp.mean(h, axis=0, keepdims=True)
    c = jnp.sum(g * wg_ref[0:1, :], axis=1, keepdims=True) + wg_ref[2:3, 0:1]
    t = jnp.sum(h * wg_ref[1:2, :], axis=1, keepdims=True) + c
    gl = jnp.where(t >= 0, t, 0.01 * t)
    gl = gl - jnp.max(gl, axis=0, keepdims=True)
    ew = jnp.exp(gl)
    a = ew / jnp.sum(ew, axis=0, keepdims=True)
    ctx = jnp.sum(a * h, axis=0, keepdims=True)
    ctx = jnp.where(ctx > 0, ctx, jnp.exp(jnp.minimum(ctx, 0.0)) - 1.0)
    zr = jax.nn.sigmoid(
        jnp.dot(ctx, rwxzr_ref[...], preferred_element_type=F32)
        + jnp.dot(g, rwhzr_ref[...], preferred_element_type=F32)
        + rbzr_ref[...]
    )
    z = zr[:, :H]
    r = zr[:, H:]
    n = jnp.tanh(
        jnp.dot(ctx, rwxn_ref[...], preferred_element_type=F32)
        + jnp.dot(r * g, rwhn_ref[...], preferred_element_type=F32)
        + rbn_ref[...]
    )
    gg = (1.0 - z) * n + z * g
    hid = jnp.maximum(
        jnp.dot(gg, w1_ref[...], preferred_element_type=F32) + b1_ref[...], 0.0
    )
    o_ref[...] = jnp.dot(hid, w2_ref[...], preferred_element_type=F32) + b2_ref[...]


def _tc_readout(h, wg3, p):
    return pl.pallas_call(
        _readout_body,
        out_shape=jax.ShapeDtypeStruct((1, 1), F32),
    )(h, wg3, p["rWx_zr"], p["rWh_zr"], p["rb_zr"][None, :], p["rWx_n"],
      p["rWh_n"], p["rb_n"][None, :], p["W1"], p["b1"][None, :], p["W2"],
      p["b2"][None, :])


def _sc_edge_body(src_hbm, dst_hbm, hm_hbm, ha_hbm, em_hbm, wab_hbm, z128_hbm,
                  c_out, ha_v, wab_v, idx_src, idx_dst, rows_v, em_v, m_v,
                  tbuf, wbuf, c_sp):
    cid = lax.axis_index("c")
    sid = lax.axis_index("s")

    pltpu.sync_copy(ha_hbm, ha_v)
    pltpu.sync_copy(wab_hbm, wab_v)
    row0 = sid * RPT
    pltpu.sync_copy(z128_hbm.at[pl.ds(row0, RPT), :], c_sp.at[pl.ds(row0, RPT), :])
    zero16 = jnp.zeros((16,), F32)
    for r_i in range(CH):
        for j_i in range(4, 8):
            m_v[r_i, pl.ds(16 * j_i, 16)] = zero16
    plsc.subcore_barrier()

    wab = [wab_v[pl.ds(16 * j, 16)] for j in range(4)]
    ii = lax.iota(jnp.int32, 16)
    ebase = (cid * NTILE + sid) * EPT

    def chunk(t, carry):
        base = ebase + t * CH
        pltpu.sync_copy(src_hbm.at[pl.ds(base, CH)], idx_src)
        pltpu.sync_copy(dst_hbm.at[pl.ds(base, CH)], idx_dst)
        pltpu.sync_copy(hm_hbm.at[idx_src], rows_v)
        pltpu.sync_copy(em_hbm.at[pl.ds(base, CH), :], em_v)
        for g in range(CH // 16):
            dst16 = idx_dst[pl.ds(g * 16, 16)]
            had = plsc.load_gather(ha_v, [dst16])
            for i in range(16):
                e = g * 16 + i
                acc = None
                for j in range(4):
                    q = rows_v[e, pl.ds(16 * j, 16)] + em_v[e, pl.ds(16 * j, 16)]
                    m = jnp.maximum(q, 0.0)
                    m_v[e, pl.ds(16 * j, 16)] = m
                    pj = m * wab[j]
                    acc = pj if acc is None else acc + pj
                tbuf[i, :] = acc
            s = None
            for j in range(16):
                col = plsc.load_gather(tbuf, [ii, jnp.full((16,), j, jnp.int32)])
                s = col if s is None else s + col
            logit = had + s
            logit = jnp.where(logit >= 0, logit, 0.01 * logit)
            w = jnp.exp(logit)
            wbuf[...] = w
            plsc.store_scatter(
                m_v, [g * 16 + ii, jnp.full((16,), H, jnp.int32)], w
            )
            for i in range(16):
                e = g * 16 + i
                ws = plsc.load_gather(wbuf, [jnp.full((16,), i, jnp.int32)])
                for j in range(4):
                    m_v[e, pl.ds(16 * j, 16)] = m_v[e, pl.ds(16 * j, 16)] * ws
        pltpu.sync_copy(m_v, c_sp.at[idx_dst], add=True)
        return carry

    lax.fori_loop(0, NCHUNK, chunk, 0)
    plsc.subcore_barrier()
    pltpu.sync_copy(c_sp.at[pl.ds(row0, RPT), :], c_out.at[cid, pl.ds(row0, RPT), :])


@functools.lru_cache(maxsize=None)
def _get_sc_edge():
  return pl.kernel(
    _sc_edge_body,
    out_type=jax.ShapeDtypeStruct((NSC, NP, 2 * H), F32),
    mesh=plsc.VectorSubcoreMesh(
        core_axis_name="c", subcore_axis_name="s", num_cores=NSC,
        num_subcores=NTILE,
    ),
    compiler_params=pltpu.CompilerParams(needs_layout_passes=False),
    scratch_types=[
        pltpu.VMEM((N,), F32),
        pltpu.VMEM((H,), F32),
        pltpu.VMEM((CH,), jnp.int32),
        pltpu.VMEM((CH,), jnp.int32),
        pltpu.VMEM((CH, 2 * H), F32),
        pltpu.VMEM((CH, H), F32),
        pltpu.VMEM((CH, 2 * H), F32),
        pltpu.VMEM((16, 16), F32),
        pltpu.VMEM((16,), F32),
        pltpu.VMEM_SHARED((NP, 2 * H), F32),
    ],
  )


def kernel(graph, node_feats, edge_feats, params):
    p = params
    src = graph[0].astype(jnp.int32)
    dst = graph[1].astype(jnp.int32)

    wmsg_top = p["W_msg"][:, :H, :]
    wem = jnp.einsum("eh,lhk->lek", p["W_edge"], p["W_msg"][:, H:, :])
    bem = (
        jnp.einsum("h,lhk->lk", p["b_edge"], p["W_msg"][:, H:, :]) + p["b_msg"]
    )
    wat = p["W_att"][:, :H, 0]
    wab = p["W_att"][:, H:, 0]
    batt = p["b_att"][:, 0]

    wmsg_top_pad = jnp.pad(wmsg_top, ((0, 0), (0, 0), (0, H)))
    zeros128 = jnp.zeros((NP, 2 * H), F32)

    h = _tc_matmul_bias(node_feats, p["W_node"], p["b_node"][None, :], 1000)

    for l in range(L):
        em = _tc_matmul_bias(edge_feats, wem[l], bem[l][None, :], 2000)
        wat_plus_b = jnp.concatenate(
            [wat[l][None, :], jnp.full((1, H), batt[l], F32)], axis=0
        )
        hm, ha = _tc_node_proj(h, wmsg_top_pad[l], wat_plus_b)
        c2 = _get_sc_edge()(src, dst, hm, ha.reshape(N), em, wab[l], zeros128)
        h = _tc_gru(c2, h, p["Wx_zr"][l], p["Wh_zr"][l], p["b_zr"][l],
                    p["Wx_n"][l], p["Wh_n"][l], p["b_n"][l])

    wg3 = jnp.concatenate(
        [p["W_gatt"][:H, 0][None, :], p["W_gatt"][H:, 0][None, :],
         jnp.full((1, H), p["b_gatt"][0], F32)], axis=0
    )
    out = _tc_readout(h, wg3, p)
    return out.reshape(1)
